# baseline (speedup 1.0000x reference)
import functools

import jax
import jax.numpy as jnp
from jax import lax
from jax.experimental import pallas as pl
from jax.experimental.pallas import tpu as pltpu
from jax.experimental.pallas import tpu_sc as plsc

_NC = 2
_NS = 16
_NW = _NC * _NS
_C = 2
_NBUF = 4


def kernel(x, emb):
    (B,) = x.shape
    V, D = emb.shape
    bpw = B // _NW
    nchunk = bpw // _C

    x3 = x.reshape(_NW, nchunk, _C).astype(jnp.int32)

    mesh = plsc.VectorSubcoreMesh(core_axis_name="c", subcore_axis_name="s")

    @functools.partial(
        pl.kernel,
        out_type=jax.ShapeDtypeStruct((B // _C, _C, D), emb.dtype),
        mesh=mesh,
        scratch_types=[
            pltpu.VMEM((nchunk, _C), jnp.int32),
        ]
        + [pltpu.VMEM((_C, D), emb.dtype) for _ in range(_NBUF)]
        + [pltpu.SemaphoreType.DMA for _ in range(2 * _NBUF)],
    )
    def gather_k(x_hbm, emb_hbm, out_hbm, idx_v, *rest):
        bufs = rest[:_NBUF]
        gsem = rest[_NBUF : 2 * _NBUF]
        wsem = rest[2 * _NBUF :]
        wid = lax.axis_index("s") * _NC + lax.axis_index("c")
        cbase = wid * nchunk
        pltpu.sync_copy(x_hbm.at[wid], idx_v)

        def fire_g(g, b):
            pltpu.async_copy(emb_hbm.at[idx_v.at[g]], bufs[b], gsem[b])

        def wait_g(g, b):
            pltpu.make_async_copy(emb_hbm.at[idx_v.at[g]], bufs[b], gsem[b]).wait()

        def fire_w(g, b):
            pltpu.async_copy(bufs[b], out_hbm.at[cbase + g], wsem[b])

        def wait_w(g, b):
            pltpu.make_async_copy(bufs[b], out_hbm.at[cbase + g], wsem[b]).wait()

        fire_g(0, 0)
        fire_g(1, 1)

        @pl.loop(0, nchunk, step=_NBUF)
        def _(j):
            for b in range(_NBUF):
                g = j + b
                bn = (b + 2) % _NBUF

                @pl.when(g >= 2)
                def _():
                    wait_w(g - 2, bn)

                @pl.when(g + 2 < nchunk)
                def _():
                    fire_g(g + 2, bn)

                wait_g(g, b)
                fire_w(g, b)

        wait_w(nchunk - 2, (nchunk - 2) % _NBUF)
        wait_w(nchunk - 1, (nchunk - 1) % _NBUF)

    out = gather_k(x3, emb)
    return out.reshape(B, D)


# speedup vs baseline: 2.6393x; 2.6393x over previous
import functools

import jax
import jax.numpy as jnp
from jax import lax
from jax.experimental import pallas as pl
from jax.experimental.pallas import tpu as pltpu
from jax.experimental.pallas import tpu_sc as plsc

_NC = 2
_NS = 16
_NW = _NC * _NS
_C = 4


def kernel(x, emb):
    (B,) = x.shape
    V, D = emb.shape
    bpw = B // _NW
    nchunk = bpw // _C

    x3 = x.reshape(_NW, nchunk, _C).astype(jnp.int32)

    mesh = plsc.VectorSubcoreMesh(core_axis_name="c", subcore_axis_name="s")

    @functools.partial(
        pl.kernel,
        out_type=jax.ShapeDtypeStruct((B, D), emb.dtype),
        mesh=mesh,
        scratch_types=[
            pltpu.VMEM((nchunk, _C), jnp.int32),
            pltpu.VMEM((_C, D), emb.dtype),
            pltpu.VMEM((_C, D), emb.dtype),
            pltpu.SemaphoreType.DMA,
            pltpu.SemaphoreType.DMA,
        ],
    )
    def gather_k(x_hbm, emb_hbm, out_hbm, idx_v, buf0, buf1, sem0, sem1):
        wid = lax.axis_index("s") * _NC + lax.axis_index("c")
        rbase = wid * bpw
        pltpu.sync_copy(x_hbm.at[wid], idx_v)

        pltpu.async_copy(emb_hbm.at[idx_v.at[0]], buf0, sem0)

        @pl.loop(0, nchunk, step=2)
        def _(j):
            pltpu.async_copy(emb_hbm.at[idx_v.at[j + 1]], buf1, sem1)
            pltpu.make_async_copy(emb_hbm.at[idx_v.at[j]], buf0, sem0).wait()
            pltpu.sync_copy(buf0, out_hbm.at[pl.ds(rbase + j * _C, _C)])

            @pl.when(j + 2 < nchunk)
            def _():
                pltpu.async_copy(emb_hbm.at[idx_v.at[j + 2]], buf0, sem0)

            pltpu.make_async_copy(emb_hbm.at[idx_v.at[j + 1]], buf1, sem1).wait()
            pltpu.sync_copy(buf1, out_hbm.at[pl.ds(rbase + (j + 1) * _C, _C)])

    return gather_k(x3, emb)
